# baseline (speedup 1.0000x reference)
import functools

import jax
import jax.numpy as jnp
from jax import lax
from jax.experimental import pallas as pl
from jax.experimental.pallas import tpu as pltpu
from jax.experimental.pallas import tpu_sc as plsc

L = 16
NW = 32
CHUNK = 128
NCHUNK = 4
TOK_W = CHUNK * NCHUNK

MULT_CUR = 36313
MULT_PREV = 27191


def _body(n_tok_row, mod, x_hbm, scale_hbm, embed_hbm, out_hbm,
          xbuf, hbuf, rows, sbuf, sem):
    wid = lax.axis_index("s") * 2 + lax.axis_index("c")
    p = pl.multiple_of(wid * TOK_W, TOK_W)
    row_workers = n_tok_row // TOK_W
    row_flag = jnp.minimum(wid & (row_workers - 1), 1)

    pltpu.sync_copy(x_hbm.at[pl.ds(p, TOK_W)], xbuf.at[pl.ds(8, TOK_W)])
    pb = pl.multiple_of(jnp.maximum(p - 8, 0), 8)
    pltpu.sync_copy(x_hbm.at[pl.ds(pb, 8)], xbuf.at[pl.ds(0, 8)])
    pltpu.sync_copy(scale_hbm, sbuf)

    iota = lax.broadcasted_iota(jnp.int32, (L,), 0)
    inv = jnp.float32(1.0) / jnp.float32(mod)
    for i in range(TOK_W // L):
        cur = xbuf[pl.ds(8 + i * L, L)]
        prev = xbuf[pl.ds(7 + i * L, L)]
        if i == 0:
            keep = jnp.where(iota == 0, row_flag, 1)
            prev = prev * keep
        v = (cur * MULT_CUR) ^ (prev * MULT_PREV)
        q = (v.astype(jnp.float32) * inv).astype(jnp.int32)
        r = v - q * mod
        r = jnp.where(r < 0, r + mod, r)
        r = jnp.where(r >= mod, r - mod, r)
        hbuf[i // (CHUNK // L), pl.ds((i % (CHUNK // L)) * L, L)] = r

    copies = []
    for j in range(NCHUNK):
        copies.append(pltpu.async_copy(
            embed_hbm.at[hbuf.at[j]], rows.at[pl.ds(j * CHUNK, CHUNK)], sem))
    for c in copies:
        c.wait()

    sv = sbuf[...]
    d = rows.shape[1]

    def mul_body(i, carry):
        r0 = i * L
        for rr in range(L):
            for c0 in range(d // L):
                sl = pl.ds(c0 * L, L)
                rows[r0 + rr, sl] = rows[r0 + rr, sl] * sv
        return carry

    lax.fori_loop(0, TOK_W // L, mul_body, 0)
    pltpu.sync_copy(rows, out_hbm.at[pl.ds(p, TOK_W)])


def kernel(x, embed, scale):
    b, s = x.shape
    v, d = embed.shape
    xf = x.reshape(-1)
    scale16 = jnp.full((L,), scale, jnp.float32)
    mesh = plsc.VectorSubcoreMesh(core_axis_name="c", subcore_axis_name="s")
    run = pl.kernel(
        functools.partial(_body, s, v - 1),
        mesh=mesh,
        compiler_params=pltpu.CompilerParams(use_tc_tiling_on_sc=False),
        out_type=jax.ShapeDtypeStruct((b * s, d), jnp.float32),
        scratch_types=[
            pltpu.VMEM((TOK_W + 8,), jnp.int32),
            pltpu.VMEM((NCHUNK, CHUNK), jnp.int32),
            pltpu.VMEM((TOK_W, d), jnp.float32),
            pltpu.VMEM((L,), jnp.float32),
            pltpu.SemaphoreType.DMA,
        ],
    )
    out = run(xf, scale16, embed)
    return out.reshape(b, s, d)


# speedup vs baseline: 1.7093x; 1.7093x over previous
import functools

import jax
import jax.numpy as jnp
from jax import lax
from jax.experimental import pallas as pl
from jax.experimental.pallas import tpu as pltpu
from jax.experimental.pallas import tpu_sc as plsc

L = 16
NW = 32
TOK_W = 512

MULT_CUR = 36313
MULT_PREV = 27191


def _hash16(cur, prev, mod, inv):
    v = (cur * MULT_CUR) ^ (prev * MULT_PREV)
    q = (v.astype(jnp.float32) * inv).astype(jnp.int32)
    r = v - q * mod
    r = jnp.where(r < 0, r + mod, r)
    r = jnp.where(r >= mod, r - mod, r)
    return r


def _body(n_tok_row, mod, x_hbm, scale_hbm, embed_hbm, out_hbm,
          xbuf, rows, sbuf, sem):
    wid = lax.axis_index("s") * 2 + lax.axis_index("c")
    p = pl.multiple_of(wid * TOK_W, TOK_W)
    row_workers = n_tok_row // TOK_W
    row_flag = jnp.minimum(wid & (row_workers - 1), 1)

    pltpu.sync_copy(x_hbm.at[pl.ds(p, TOK_W)], xbuf.at[pl.ds(8, TOK_W)])
    pb = pl.multiple_of(jnp.maximum(p - 8, 0), 8)
    pltpu.sync_copy(x_hbm.at[pl.ds(pb, 8)], xbuf.at[pl.ds(0, 8)])
    pltpu.sync_copy(scale_hbm, sbuf)

    iota = lax.broadcasted_iota(jnp.int32, (L,), 0)
    inv = jnp.float32(1.0) / jnp.float32(mod)

    def fire16(k, hv):
        for j in range(L):
            pltpu.async_copy(embed_hbm.at[hv[j]], rows.at[k * L + j], sem)

    cur = xbuf[pl.ds(8, L)]
    prev = xbuf[pl.ds(7, L)]
    keep = jnp.where(iota == 0, row_flag, 1)
    fire16(0, _hash16(cur, prev * keep, mod, inv))

    def gather_body(k, carry):
        cur = xbuf[pl.ds(8 + k * L, L)]
        prev = xbuf[pl.ds(7 + k * L, L)]
        fire16(k, _hash16(cur, prev, mod, inv))
        return carry

    lax.fori_loop(1, TOK_W // L, gather_body, 0)

    pltpu.make_async_copy(embed_hbm.at[pl.ds(0, TOK_W)], rows, sem).wait()

    sv = sbuf[...]
    d = rows.shape[1]

    def mul_body(i, carry):
        r0 = i * L
        for rr in range(L):
            for c0 in range(d // L):
                sl = pl.ds(c0 * L, L)
                rows[r0 + rr, sl] = rows[r0 + rr, sl] * sv
        return carry

    lax.fori_loop(0, TOK_W // L, mul_body, 0)
    pltpu.sync_copy(rows, out_hbm.at[pl.ds(p, TOK_W)])


def kernel(x, embed, scale):
    b, s = x.shape
    v, d = embed.shape
    xf = x.reshape(-1)
    scale16 = jnp.full((L,), scale, jnp.float32)
    mesh = plsc.VectorSubcoreMesh(core_axis_name="c", subcore_axis_name="s")
    run = pl.kernel(
        functools.partial(_body, s, v - 1),
        mesh=mesh,
        out_type=jax.ShapeDtypeStruct((b * s, d), jnp.float32),
        scratch_types=[
            pltpu.VMEM((TOK_W + 8,), jnp.int32),
            pltpu.VMEM((TOK_W, d), jnp.float32),
            pltpu.VMEM((L,), jnp.float32),
            pltpu.SemaphoreType.DMA,
        ],
    )
    out = run(xf, scale16, embed)
    return out.reshape(b, s, d)


# speedup vs baseline: 2.5527x; 1.4934x over previous
import functools

import jax
import jax.numpy as jnp
from jax import lax
from jax.experimental import pallas as pl
from jax.experimental.pallas import tpu as pltpu
from jax.experimental.pallas import tpu_sc as plsc

L = 16
NW = 32
TOK_W = 512

MULT_CUR = 36313
MULT_PREV = 27191


def _hash16(cur, prev, mod, inv):
    v = (cur * MULT_CUR) ^ (prev * MULT_PREV)
    q = (v.astype(jnp.float32) * inv).astype(jnp.int32)
    r = v - q * mod
    r = jnp.where(r < 0, r + mod, r)
    r = jnp.where(r >= mod, r - mod, r)
    return r


def _body(n_tok_row, mod, x_hbm, scale_hbm, embed_hbm, out_hbm,
          xbuf, rows, sbuf, sem):
    wid = lax.axis_index("s") * 2 + lax.axis_index("c")
    p = pl.multiple_of(wid * TOK_W, TOK_W)
    row_workers = n_tok_row // TOK_W
    row_flag = jnp.minimum(wid & (row_workers - 1), 1)

    pltpu.sync_copy(x_hbm.at[pl.ds(p, TOK_W)], xbuf.at[pl.ds(8, TOK_W)])
    pb = pl.multiple_of(jnp.maximum(p - 8, 0), 8)
    pltpu.sync_copy(x_hbm.at[pl.ds(pb, 8)], xbuf.at[pl.ds(0, 8)])
    pltpu.sync_copy(scale_hbm, sbuf)

    iota = lax.broadcasted_iota(jnp.int32, (L,), 0)
    inv = jnp.float32(1.0) / jnp.float32(mod)

    def fire16(k, hv):
        for j in range(L):
            h = hv[j]
            pltpu.async_copy(embed_hbm.at[h >> 3, h & 7], rows.at[k * L + j],
                             sem)

    cur = xbuf[pl.ds(8, L)]
    prev = xbuf[pl.ds(7, L)]
    keep = jnp.where(iota == 0, row_flag, 1)
    fire16(0, _hash16(cur, prev * keep, mod, inv))

    def gather_body(k, carry):
        cur = xbuf[pl.ds(8 + k * L, L)]
        prev = xbuf[pl.ds(7 + k * L, L)]
        fire16(k, _hash16(cur, prev, mod, inv))
        return carry

    lax.fori_loop(1, TOK_W // L, gather_body, 0)

    pltpu.make_async_copy(out_hbm.at[pl.ds(0, TOK_W)], rows, sem).wait()

    sv = sbuf[...]
    d = rows.shape[1]

    def mul_body(i, carry):
        r0 = i * L
        for rr in range(L):
            for c0 in range(d // L):
                sl = pl.ds(c0 * L, L)
                rows[r0 + rr, sl] = rows[r0 + rr, sl] * sv
        return carry

    lax.fori_loop(0, TOK_W // L, mul_body, 0)
    pltpu.sync_copy(rows, out_hbm.at[pl.ds(p, TOK_W)])


def kernel(x, embed, scale):
    b, s = x.shape
    v, d = embed.shape
    xf = x.reshape(-1)
    scale16 = jnp.full((L,), scale, jnp.float32)
    mesh = plsc.VectorSubcoreMesh(core_axis_name="c", subcore_axis_name="s")
    run = pl.kernel(
        functools.partial(_body, s, v - 1),
        mesh=mesh,
        out_type=jax.ShapeDtypeStruct((b * s, d), jnp.float32),
        scratch_types=[
            pltpu.VMEM((TOK_W + 8,), jnp.int32),
            pltpu.VMEM((TOK_W, d), jnp.float32),
            pltpu.VMEM((L,), jnp.float32),
            pltpu.SemaphoreType.DMA,
        ],
    )
    out = run(xf, scale16, embed.reshape(v // 8, 8, d))
    return out.reshape(b, s, d)
